# baseline (speedup 1.0000x reference)
import functools

import jax
import jax.numpy as jnp
from jax import lax
from jax.experimental import pallas as pl
from jax.experimental.pallas import tpu as pltpu
from jax.experimental.pallas import tpu_sc as plsc

BATCH = 16384
EMB = 32
RPS = 4
SUP = 250000
NC, NS, L = 2, 16, 16
NW = NC * NS
BPW = BATCH // NW
CHUNK = 128
NCH = BPW // CHUNK
GPC = CHUNK // L

_mesh = plsc.VectorSubcoreMesh(core_axis_name="c", subcore_axis_name="s")


@functools.partial(
    pl.kernel,
    mesh=_mesh,
    out_type=jax.ShapeDtypeStruct((BATCH,), jnp.float32),
    scratch_types=[
        pltpu.VMEM((BPW,), jnp.int32),
        pltpu.VMEM((BPW,), jnp.int32),
        pltpu.VMEM((BPW,), jnp.int32),
        pltpu.VMEM((BPW,), jnp.int32),
        pltpu.VMEM((CHUNK, RPS * EMB), jnp.float32),
        pltpu.VMEM((CHUNK, RPS * EMB), jnp.float32),
        pltpu.VMEM((BPW,), jnp.float32),
        pltpu.SemaphoreType.DMA,
    ],
)
def _sc_scores(users_hbm, items_hbm, uemb_hbm, iemb_hbm, out_hbm,
               uidx_v, iidx_v, usup_v, isup_v,
               urows_v, irows_v, score_v, sem):
    wid = lax.axis_index("s") * NC + lax.axis_index("c")
    base = wid * BPW
    pltpu.sync_copy(users_hbm.at[pl.ds(base, BPW)], uidx_v)
    pltpu.sync_copy(items_hbm.at[pl.ds(base, BPW)], iidx_v)
    for v in range(BPW // L):
        sl = pl.ds(v * L, L)
        usup_v[sl] = lax.shift_right_logical(uidx_v[sl], 2)
        isup_v[sl] = lax.shift_right_logical(iidx_v[sl], 2)

    lane = lax.iota(jnp.int32, L)
    bitrev = [int(f"{j:04b}"[::-1], 2) for j in range(L)]

    _dnums = lax.GatherDimensionNumbers(
        offset_dims=(), collapsed_slice_dims=(0,), start_index_map=(0,))

    def permute(x, idx):
        return lax.gather(x, idx[:, None], dimension_numbers=_dnums,
                          slice_sizes=(1,),
                          mode=lax.GatherScatterMode.PROMISE_IN_BOUNDS)

    def chunk_body(c, _):
        cu = pltpu.async_copy(
            uemb_hbm.at[usup_v.at[pl.ds(c * CHUNK, CHUNK)]], urows_v, sem)
        ci = pltpu.async_copy(
            iemb_hbm.at[isup_v.at[pl.ds(c * CHUNK, CHUNK)]], irows_v, sem)
        cu.wait()
        ci.wait()

        def group(g, _):
            r0 = c * CHUNK + g * L
            usub = (uidx_v[pl.ds(r0, L)] & 3) * EMB
            isub = (iidx_v[pl.ds(r0, L)] & 3) * EMB
            vs = []
            for j in range(L):
                rl = g * L + bitrev[j]
                uo = usub[bitrev[j]]
                io = isub[bitrev[j]]
                u0 = urows_v[rl, pl.ds(uo, L)]
                u1 = urows_v[rl, pl.ds(uo + L, L)]
                i0 = irows_v[rl, pl.ds(io, L)]
                i1 = irows_v[rl, pl.ds(io + L, L)]
                vs.append(u0 * i0 + u1 * i1)
            for d in (8, 4, 2, 1):
                perm = lane ^ d
                keep = (lane & d) == 0
                nxt = []
                for k in range(len(vs) // 2):
                    a, b = vs[2 * k], vs[2 * k + 1]
                    fa = a + permute(a, perm)
                    fb = b + permute(b, perm)
                    nxt.append(jnp.where(keep, fa, fb))
                vs = nxt
            score_v[pl.ds(r0, L)] = vs[0]
            return 0

        lax.fori_loop(0, GPC, group, 0)
        return 0

    lax.fori_loop(0, NCH, chunk_body, 0)
    pltpu.sync_copy(score_v, out_hbm.at[pl.ds(base, BPW)])


def kernel(users, items, user_emb, item_emb):
    return _sc_scores(users.astype(jnp.int32), items.astype(jnp.int32),
                      user_emb.reshape(SUP, RPS * EMB),
                      item_emb.reshape(SUP, RPS * EMB))


# speedup vs baseline: 3.3339x; 3.3339x over previous
import functools

import jax
import jax.numpy as jnp
from jax import lax
from jax.experimental import pallas as pl
from jax.experimental.pallas import tpu as pltpu
from jax.experimental.pallas import tpu_sc as plsc

BATCH = 16384
EMB = 32
NROW = 1000000
NC, NS, L = 2, 16, 16
DPC = EMB // NC
BPS = BATCH // NS
NV = BPS // L
SEG = 62464
LASTSEG = NROW - (NS - 1) * SEG

_mesh = plsc.VectorSubcoreMesh(core_axis_name="c", subcore_axis_name="s")


@functools.partial(
    pl.kernel,
    mesh=_mesh,
    out_type=jax.ShapeDtypeStruct((NC, BATCH), jnp.float32),
    scratch_types=[
        pltpu.VMEM((BPS,), jnp.int32),
        pltpu.VMEM((BPS,), jnp.int32),
        pltpu.VMEM((BPS,), jnp.float32),
        pltpu.VMEM((BPS,), jnp.float32),
        pltpu.VMEM((BPS,), jnp.float32),
        pltpu.VMEM_SHARED((1, NROW), jnp.float32),
        pltpu.SemaphoreType.DMA,
    ],
)
def _sc_scores(users_hbm, items_hbm, uembt_hbm, iembt_hbm, out_hbm,
               uidx_v, iidx_v, ud_v, id_v, acc_v, row_sh, sem):
    cid = lax.axis_index("c")
    sid = lax.axis_index("s")
    base = sid * BPS
    pltpu.sync_copy(users_hbm.at[pl.ds(base, BPS)], uidx_v)
    pltpu.sync_copy(items_hbm.at[pl.ds(base, BPS)], iidx_v)

    seg_off = sid * SEG

    def stage(tbl_hbm, d):
        pltpu.sync_copy(
            tbl_hbm.at[pl.ds(d, 1), pl.ds(seg_off, SEG)],
            row_sh.at[pl.ds(0, 1), pl.ds(seg_off, SEG)],
        )

        @pl.when(sid == NS - 1)
        def _():
            pltpu.sync_copy(
                tbl_hbm.at[pl.ds(d, 1), pl.ds(NS * SEG, NROW - NS * SEG)],
                row_sh.at[pl.ds(0, 1), pl.ds(NS * SEG, NROW - NS * SEG)],
            )

    def gather(idx_v, dst_v):
        pltpu.async_copy(row_sh.at[0].at[idx_v], dst_v, sem).wait()

    for v in range(NV):
        acc_v[pl.ds(v * L, L)] = jnp.zeros((L,), jnp.float32)

    for dl in range(DPC):
        d = cid * DPC + dl
        plsc.subcore_barrier()
        stage(uembt_hbm, d)
        plsc.subcore_barrier()
        gather(uidx_v, ud_v)
        plsc.subcore_barrier()
        stage(iembt_hbm, d)
        plsc.subcore_barrier()
        gather(iidx_v, id_v)
        for v in range(NV):
            sl = pl.ds(v * L, L)
            acc_v[sl] = acc_v[sl] + ud_v[sl] * id_v[sl]

    pltpu.sync_copy(acc_v, out_hbm.at[cid, pl.ds(base, BPS)])


def kernel(users, items, user_emb, item_emb):
    partials = _sc_scores(users.astype(jnp.int32), items.astype(jnp.int32),
                          user_emb.T, item_emb.T)
    return partials[0] + partials[1]


# speedup vs baseline: 3.9518x; 1.1854x over previous
import functools

import jax
import jax.numpy as jnp
from jax import lax
from jax.experimental import pallas as pl
from jax.experimental.pallas import tpu as pltpu
from jax.experimental.pallas import tpu_sc as plsc

BATCH = 16384
EMB = 32
NROW = 1000000
NC, NS, L = 2, 16, 16
DPC = EMB // NC
BPS = BATCH // NS
NV = BPS // L
SEG = 62464
TAIL = NROW - NS * SEG

_mesh = plsc.VectorSubcoreMesh(core_axis_name="c", subcore_axis_name="s")


@functools.partial(
    pl.kernel,
    mesh=_mesh,
    out_type=jax.ShapeDtypeStruct((NC, BATCH), jnp.float32),
    scratch_types=[
        pltpu.VMEM((BPS,), jnp.int32),
        pltpu.VMEM((BPS,), jnp.int32),
        pltpu.VMEM((BPS,), jnp.float32),
        pltpu.VMEM((BPS,), jnp.float32),
        pltpu.VMEM((BPS,), jnp.float32),
        pltpu.VMEM((2 * DPC, TAIL), jnp.float32),
        pltpu.VMEM_SHARED((1, NROW), jnp.float32),
        pltpu.SemaphoreType.DMA,
    ],
)
def _sc_scores(users_hbm, items_hbm, uembt_hbm, iembt_hbm, out_hbm,
               uidx_v, iidx_v, ud_v, id_v, acc_v, tails_v,
               row_sh, sem):
    cid = lax.axis_index("c")
    sid = lax.axis_index("s")
    base = sid * BPS
    pltpu.sync_copy(users_hbm.at[pl.ds(base, BPS)], uidx_v)
    pltpu.sync_copy(items_hbm.at[pl.ds(base, BPS)], iidx_v)

    seg_off = sid * SEG

    @pl.when(sid == 0)
    def _():
        pltpu.sync_copy(
            uembt_hbm.at[pl.ds(cid * DPC, DPC), pl.ds(NS * SEG, TAIL)],
            tails_v.at[pl.ds(0, DPC)])
        pltpu.sync_copy(
            iembt_hbm.at[pl.ds(cid * DPC, DPC), pl.ds(NS * SEG, TAIL)],
            tails_v.at[pl.ds(DPC, DPC)])

    for v in range(NV):
        acc_v[pl.ds(v * L, L)] = jnp.zeros((L,), jnp.float32)

    def stage_and_gather(tbl_hbm, dl, d, tail_row, idx_v, dst_v):
        plsc.subcore_barrier()
        pltpu.sync_copy(
            tbl_hbm.at[pl.ds(d, 1), pl.ds(seg_off, SEG)],
            row_sh.at[pl.ds(0, 1), pl.ds(seg_off, SEG)])

        @pl.when(sid == 0)
        def _():
            pltpu.sync_copy(tails_v.at[pl.ds(tail_row + dl, 1)],
                            row_sh.at[pl.ds(0, 1), pl.ds(NS * SEG, TAIL)])

        plsc.subcore_barrier()
        pltpu.async_copy(row_sh.at[0].at[idx_v], dst_v, sem).wait()

    for dl in range(DPC):
        d = cid * DPC + dl
        stage_and_gather(uembt_hbm, dl, d, 0, uidx_v, ud_v)
        stage_and_gather(iembt_hbm, dl, d, DPC, iidx_v, id_v)
        for v in range(NV):
            sl = pl.ds(v * L, L)
            acc_v[sl] = acc_v[sl] + ud_v[sl] * id_v[sl]

    pltpu.sync_copy(acc_v, out_hbm.at[cid, pl.ds(base, BPS)])


def kernel(users, items, user_emb, item_emb):
    partials = _sc_scores(users.astype(jnp.int32), items.astype(jnp.int32),
                          user_emb.T, item_emb.T)
    return partials[0] + partials[1]


# speedup vs baseline: 3.9622x; 1.0026x over previous
import functools

import jax
import jax.numpy as jnp
from jax import lax
from jax.experimental import pallas as pl
from jax.experimental.pallas import tpu as pltpu
from jax.experimental.pallas import tpu_sc as plsc

BATCH = 16384
EMB = 32
NROW = 1000000
NC, NS, L = 2, 16, 16
DPC = EMB // NC
BPS = BATCH // NS
NV = BPS // L
SEG = 62464
TAIL = NROW - NS * SEG

_mesh = plsc.VectorSubcoreMesh(core_axis_name="c", subcore_axis_name="s")


@functools.partial(
    pl.kernel,
    mesh=_mesh,
    out_type=jax.ShapeDtypeStruct((NC, BATCH), jnp.float32),
    scratch_types=[
        pltpu.VMEM((BPS,), jnp.int32),
        pltpu.VMEM((BPS,), jnp.int32),
        pltpu.VMEM((BPS,), jnp.float32),
        pltpu.VMEM((BPS,), jnp.float32),
        pltpu.VMEM((BPS,), jnp.float32),
        pltpu.VMEM((2 * DPC, TAIL), jnp.float32),
        pltpu.VMEM_SHARED((1, NROW), jnp.float32),
        pltpu.SemaphoreType.DMA,
    ],
)
def _sc_scores(users_hbm, items_hbm, uembt_hbm, iembt_hbm, out_hbm,
               uidx_v, iidx_v, ud_v, id_v, acc_v, tails_v,
               row_sh, sem):
    cid = lax.axis_index("c")
    sid = lax.axis_index("s")
    base = sid * BPS
    pltpu.sync_copy(users_hbm.at[pl.ds(base, BPS)], uidx_v)
    pltpu.sync_copy(items_hbm.at[pl.ds(base, BPS)], iidx_v)

    seg_off = sid * SEG

    @pl.when(sid == 0)
    def _():
        pltpu.sync_copy(
            uembt_hbm.at[pl.ds(cid * DPC, DPC), pl.ds(NS * SEG, TAIL)],
            tails_v.at[pl.ds(0, DPC)])
        pltpu.sync_copy(
            iembt_hbm.at[pl.ds(cid * DPC, DPC), pl.ds(NS * SEG, TAIL)],
            tails_v.at[pl.ds(DPC, DPC)])

    for v in range(NV):
        acc_v[pl.ds(v * L, L)] = jnp.zeros((L,), jnp.float32)

    def stage_and_gather(tbl_hbm, dl, d, tail_row, idx_v, dst_v):
        plsc.subcore_barrier()
        h = SEG // 2
        c0 = pltpu.async_copy(
            tbl_hbm.at[pl.ds(d, 1), pl.ds(seg_off, h)],
            row_sh.at[pl.ds(0, 1), pl.ds(seg_off, h)], sem)
        c1 = pltpu.async_copy(
            tbl_hbm.at[pl.ds(d, 1), pl.ds(seg_off + h, h)],
            row_sh.at[pl.ds(0, 1), pl.ds(seg_off + h, h)], sem)
        c0.wait()
        c1.wait()

        @pl.when(sid == 0)
        def _():
            pltpu.sync_copy(tails_v.at[pl.ds(tail_row + dl, 1)],
                            row_sh.at[pl.ds(0, 1), pl.ds(NS * SEG, TAIL)])

        plsc.subcore_barrier()
        pltpu.async_copy(row_sh.at[0].at[idx_v], dst_v, sem).wait()

    for dl in range(DPC):
        d = cid * DPC + dl
        stage_and_gather(uembt_hbm, dl, d, 0, uidx_v, ud_v)
        stage_and_gather(iembt_hbm, dl, d, DPC, iidx_v, id_v)
        for v in range(NV):
            sl = pl.ds(v * L, L)
            acc_v[sl] = acc_v[sl] + ud_v[sl] * id_v[sl]

    pltpu.sync_copy(acc_v, out_hbm.at[cid, pl.ds(base, BPS)])


def kernel(users, items, user_emb, item_emb):
    partials = _sc_scores(users.astype(jnp.int32), items.astype(jnp.int32),
                          user_emb.T, item_emb.T)
    return partials[0] + partials[1]
